# baseline (speedup 1.0000x reference)
import functools

import jax
import jax.numpy as jnp
from jax import lax
from jax.experimental import pallas as pl
from jax.experimental.pallas import tpu as pltpu
from jax.experimental.pallas import tpu_sc as plsc

N = 10000
D = 128
E = 320000
NC = 2
NS = 16
NW = NC * NS
CH = 80
DEG_ROWS = E // CH
DEG_TROWS = DEG_ROWS // NS
RPS = N // NS

ACH = 128
AROWS = 40
EPW = (E // 2) // NW
APAD = AROWS * ACH - EPW
NPAD = N + 16
NPS = NPAD // NS

_MESH = plsc.VectorSubcoreMesh(core_axis_name="c", subcore_axis_name="s")


def _deg_body(dsts_hbm, ones_hbm, zeros_hbm, deg_hbm, dv, ov, acc, sem):
    c = lax.axis_index("c")
    s = lax.axis_index("s")
    pltpu.sync_copy(ones_hbm, ov)
    pltpu.sync_copy(dsts_hbm.at[c, pl.ds(s * DEG_TROWS, DEG_TROWS)], dv)
    pltpu.sync_copy(zeros_hbm, acc.at[pl.ds(s * RPS, RPS)])
    plsc.subcore_barrier()

    def group(t, _):
        for j in range(10):
            pltpu.async_copy(ov, acc.at[dv.at[t * 10 + j]], sem, add=True)
        for j in range(10):
            pltpu.make_async_copy(ov, acc.at[dv.at[t * 10 + j]], sem).wait()
        return 0

    lax.fori_loop(0, DEG_TROWS // 10, group, 0)
    plsc.subcore_barrier()
    pltpu.sync_copy(acc.at[pl.ds(s * RPS, RPS)],
                    deg_hbm.at[c, pl.ds(s * RPS, RPS)])


_SC_PARAMS = pltpu.CompilerParams(use_tc_tiling_on_sc=False)

_deg_kernel = pl.kernel(
    _deg_body,
    out_type=jax.ShapeDtypeStruct((2, N, 16), jnp.float32),
    mesh=_MESH,
    compiler_params=_SC_PARAMS,
    scratch_types=[
        pltpu.VMEM((DEG_TROWS, CH), jnp.int32),
        pltpu.VMEM((CH, 16), jnp.float32),
        pltpu.VMEM_SHARED((N, 16), jnp.float32),
        pltpu.SemaphoreType.DMA,
    ],
)


def _agg_body(src_hbm, dst_hbm, g_hbm, out_hbm,
              si, di, rows0, rows1, acc, sem0, sem1, ssem0, ssem1):
    c = lax.axis_index("c")
    s = lax.axis_index("s")
    w = c * NS + s

    def _zr(r, _):
        for j in range(D // 16):
            rows0[r, pl.ds(j * 16, 16)] = jnp.zeros((16,), jnp.float32)
        return 0

    lax.fori_loop(0, ACH, _zr, 0)
    for t in range(4):
        pltpu.sync_copy(rows0, acc.at[pl.ds(s * NPS + t * ACH, ACH)])
    pltpu.sync_copy(rows0.at[pl.ds(0, NPS - 4 * ACH)],
                    acc.at[pl.ds(s * NPS + 4 * ACH, NPS - 4 * ACH)])
    pltpu.sync_copy(src_hbm.at[pl.ds(w * AROWS, AROWS)], si)
    pltpu.sync_copy(dst_hbm.at[pl.ds(w * AROWS, AROWS)], di)
    plsc.subcore_barrier()

    pltpu.async_copy(g_hbm.at[si.at[0]], rows0, sem0)
    pltpu.async_copy(g_hbm.at[si.at[1]], rows1, sem1)

    def body(i, _):
        k = 2 * i
        pltpu.make_async_copy(g_hbm.at[si.at[k]], rows0, sem0).wait()
        pltpu.async_copy(rows0, acc.at[di.at[k]], ssem0, add=True)
        pltpu.make_async_copy(g_hbm.at[si.at[k + 1]], rows1, sem1).wait()
        pltpu.async_copy(rows1, acc.at[di.at[k + 1]], ssem1, add=True)
        pltpu.make_async_copy(rows0, acc.at[di.at[k]], ssem0).wait()

        @pl.when(k + 2 < AROWS)
        def _fire0():
            pltpu.async_copy(g_hbm.at[si.at[k + 2]], rows0, sem0)

        pltpu.make_async_copy(rows1, acc.at[di.at[k + 1]], ssem1).wait()

        @pl.when(k + 3 < AROWS)
        def _fire1():
            pltpu.async_copy(g_hbm.at[si.at[k + 3]], rows1, sem1)

        return 0

    lax.fori_loop(0, AROWS // 2, body, 0)
    plsc.subcore_barrier()
    pltpu.sync_copy(acc.at[pl.ds(s * NPS, NPS)],
                    out_hbm.at[c, pl.ds(s * NPS, NPS)])


_agg_kernel = pl.kernel(
    _agg_body,
    out_type=jax.ShapeDtypeStruct((2, NPAD, D), jnp.float32),
    mesh=_MESH,
    compiler_params=_SC_PARAMS,
    scratch_types=[
        pltpu.VMEM((AROWS, ACH), jnp.int32),
        pltpu.VMEM((AROWS, ACH), jnp.int32),
        pltpu.VMEM((ACH, D), jnp.float32),
        pltpu.VMEM((ACH, D), jnp.float32),
        pltpu.VMEM_SHARED((NPAD, D), jnp.float32),
        pltpu.SemaphoreType.DMA,
        pltpu.SemaphoreType.DMA,
        pltpu.SemaphoreType.DMA,
        pltpu.SemaphoreType.DMA,
    ],
)


R = 1000


def _tc_b_body(x_ref, w_ref, deg_ref, g_ref):
    h = jnp.dot(x_ref[...], w_ref[...], preferred_element_type=jnp.float32)
    dis = lax.rsqrt(deg_ref[:, 0:1] + 1.0)
    g_ref[...] = dis * h


def _tc_d_body(pa_ref, pb_ref, g0_ref, deg0_ref, b0_ref, w1_ref, deg1_ref,
               g1_ref):
    ssum = (pa_ref[0] + pa_ref[1]) + (pb_ref[0] + pb_ref[1]) + g0_ref[...]
    dis0 = lax.rsqrt(deg0_ref[:, 0:1] + 1.0)
    a = jnp.maximum(dis0 * ssum + b0_ref[...], 0.0)
    h1 = jnp.dot(a, w1_ref[...], preferred_element_type=jnp.float32)
    dis1 = lax.rsqrt(deg1_ref[:, 0:1] + 1.0)
    g1_ref[...] = dis1 * h1


def _tc_f_body(pa_ref, pb_ref, g1_ref, deg1_ref, b1_ref, o_ref):
    dis1 = lax.rsqrt(deg1_ref[:, 0:1] + 1.0)
    v = dis1 * ((pa_ref[0] + pa_ref[1]) + (pb_ref[0] + pb_ref[1])
                + g1_ref[...]) + b1_ref[...]
    nrm = jnp.sqrt(jnp.sum(v * v, axis=1, keepdims=True))
    o_ref[...] = v / jnp.maximum(nrm, 1e-12)


def _row_spec(i):
    return (i, 0)


_tc_b = pl.pallas_call(
    _tc_b_body,
    grid=(N // R,),
    in_specs=[
        pl.BlockSpec((R, D), _row_spec),
        pl.BlockSpec((D, D), lambda i: (0, 0)),
        pl.BlockSpec((R, 16), _row_spec),
    ],
    out_specs=pl.BlockSpec((R, D), _row_spec),
    out_shape=jax.ShapeDtypeStruct((N, D), jnp.float32),
)

_part_spec = pl.BlockSpec((2, R, D), lambda i: (0, i, 0))

_tc_d = pl.pallas_call(
    _tc_d_body,
    grid=(N // R,),
    in_specs=[
        _part_spec,
        _part_spec,
        pl.BlockSpec((R, D), _row_spec),
        pl.BlockSpec((R, 16), _row_spec),
        pl.BlockSpec((1, D), lambda i: (0, 0)),
        pl.BlockSpec((D, D), lambda i: (0, 0)),
        pl.BlockSpec((R, 16), _row_spec),
    ],
    out_specs=pl.BlockSpec((R, D), _row_spec),
    out_shape=jax.ShapeDtypeStruct((N, D), jnp.float32),
)

_tc_f = pl.pallas_call(
    _tc_f_body,
    grid=(N // R,),
    in_specs=[
        _part_spec,
        _part_spec,
        pl.BlockSpec((R, D), _row_spec),
        pl.BlockSpec((R, 16), _row_spec),
        pl.BlockSpec((1, D), lambda i: (0, 0)),
    ],
    out_specs=pl.BlockSpec((R, D), _row_spec),
    out_shape=jax.ShapeDtypeStruct((N, D), jnp.float32),
)


@jax.jit
def kernel(x, edge_index_list, W0, b0, W1, b1):
    ei = edge_index_list.astype(jnp.int32)

    def pad_idx(a, fill):
        a2 = a.reshape(NW, EPW)
        pad = jnp.full((NW, APAD), fill, jnp.int32)
        return jnp.concatenate([a2, pad], axis=1).reshape(NW * AROWS, ACH)

    h = E // 2
    halves = [
        (pad_idx(ei[l, 0, i * h:(i + 1) * h], 0),
         pad_idx(ei[l, 1, i * h:(i + 1) * h], N))
        for l in range(2) for i in range(2)
    ]

    dsts_deg = jnp.stack([ei[0, 1], ei[1, 1]]).reshape(2, NS * DEG_TROWS, CH)
    ones_in = jnp.ones((CH, 16), jnp.float32)
    zeros16 = jnp.zeros((RPS, 16), jnp.float32)

    deg = _deg_kernel(dsts_deg, ones_in, zeros16)
    g0 = _tc_b(x, W0, deg[0])
    p0a = _agg_kernel(halves[0][0], halves[0][1], g0)
    p0b = _agg_kernel(halves[1][0], halves[1][1], g0)
    g1 = _tc_d(p0a, p0b, g0, deg[0], b0.reshape(1, D), W1, deg[1])
    p1a = _agg_kernel(halves[2][0], halves[2][1], g1)
    p1b = _agg_kernel(halves[3][0], halves[3][1], g1)
    return _tc_f(p1a, p1b, g1, deg[1], b1.reshape(1, D))


# speedup vs baseline: 1.0643x; 1.0643x over previous
import functools

import jax
import jax.numpy as jnp
from jax import lax
from jax.experimental import pallas as pl
from jax.experimental.pallas import tpu as pltpu
from jax.experimental.pallas import tpu_sc as plsc

N = 10000
D = 128
E = 320000
NC = 2
NS = 16
NW = NC * NS
CH = 80
DEG_ROWS = E // CH
DEG_TROWS = DEG_ROWS // NS
RPS = N // NS

ACH = 128
AROWS = 40
EPW = (E // 2) // NW
APAD = AROWS * ACH - EPW
NPAD = N + 16
NPS = NPAD // NS

_MESH = plsc.VectorSubcoreMesh(core_axis_name="c", subcore_axis_name="s")


def _deg_body(dsts_hbm, ones_hbm, zeros_hbm, deg_hbm, dv, ov, acc, sem):
    c = lax.axis_index("c")
    s = lax.axis_index("s")
    pltpu.sync_copy(ones_hbm, ov)
    pltpu.sync_copy(dsts_hbm.at[c, pl.ds(s * DEG_TROWS, DEG_TROWS)], dv)
    pltpu.sync_copy(zeros_hbm, acc.at[pl.ds(s * RPS, RPS)])
    plsc.subcore_barrier()

    def group(t, _):
        for j in range(10):
            pltpu.async_copy(ov, acc.at[dv.at[t * 10 + j]], sem, add=True)
        for j in range(10):
            pltpu.make_async_copy(ov, acc.at[dv.at[t * 10 + j]], sem).wait()
        return 0

    lax.fori_loop(0, DEG_TROWS // 10, group, 0)
    plsc.subcore_barrier()
    pltpu.sync_copy(acc.at[pl.ds(s * RPS, RPS)],
                    deg_hbm.at[c, pl.ds(s * RPS, RPS)])


_SC_PARAMS = pltpu.CompilerParams(use_tc_tiling_on_sc=False)

_deg_kernel = pl.kernel(
    _deg_body,
    out_type=jax.ShapeDtypeStruct((2, N, 16), jnp.float32),
    mesh=_MESH,
    compiler_params=_SC_PARAMS,
    scratch_types=[
        pltpu.VMEM((DEG_TROWS, CH), jnp.int32),
        pltpu.VMEM((CH, 16), jnp.float32),
        pltpu.VMEM_SHARED((N, 16), jnp.float32),
        pltpu.SemaphoreType.DMA,
    ],
)


def _agg_body(src_hbm, dst_hbm, g_hbm, out_hbm,
              si, di, rows0, rows1, acc, sem0, sem1):
    c = lax.axis_index("c")
    s = lax.axis_index("s")
    w = c * NS + s

    def _zr(r, _):
        for j in range(D // 16):
            rows0[r, pl.ds(j * 16, 16)] = jnp.zeros((16,), jnp.float32)
        return 0

    lax.fori_loop(0, ACH, _zr, 0)
    for t in range(4):
        pltpu.sync_copy(rows0, acc.at[pl.ds(s * NPS + t * ACH, ACH)])
    pltpu.sync_copy(rows0.at[pl.ds(0, NPS - 4 * ACH)],
                    acc.at[pl.ds(s * NPS + 4 * ACH, NPS - 4 * ACH)])
    pltpu.sync_copy(src_hbm.at[pl.ds(w * AROWS, AROWS)], si)
    pltpu.sync_copy(dst_hbm.at[pl.ds(w * AROWS, AROWS)], di)
    plsc.subcore_barrier()

    pltpu.async_copy(g_hbm.at[si.at[0]], rows0, sem0)

    def body(i, _):
        k = 2 * i
        pltpu.async_copy(g_hbm.at[si.at[k + 1]], rows1, sem1)
        pltpu.make_async_copy(g_hbm.at[si.at[k]], rows0, sem0).wait()
        pltpu.sync_copy(rows0, acc.at[di.at[k]], add=True)

        @pl.when(k + 2 < AROWS)
        def _fire():
            pltpu.async_copy(g_hbm.at[si.at[k + 2]], rows0, sem0)

        pltpu.make_async_copy(g_hbm.at[si.at[k + 1]], rows1, sem1).wait()
        pltpu.sync_copy(rows1, acc.at[di.at[k + 1]], add=True)
        return 0

    lax.fori_loop(0, AROWS // 2, body, 0)
    plsc.subcore_barrier()
    pltpu.sync_copy(acc.at[pl.ds(s * NPS, NPS)],
                    out_hbm.at[c, pl.ds(s * NPS, NPS)])


_agg_kernel = pl.kernel(
    _agg_body,
    out_type=jax.ShapeDtypeStruct((2, NPAD, D), jnp.float32),
    mesh=_MESH,
    compiler_params=_SC_PARAMS,
    scratch_types=[
        pltpu.VMEM((AROWS, ACH), jnp.int32),
        pltpu.VMEM((AROWS, ACH), jnp.int32),
        pltpu.VMEM((ACH, D), jnp.float32),
        pltpu.VMEM((ACH, D), jnp.float32),
        pltpu.VMEM_SHARED((NPAD, D), jnp.float32),
        pltpu.SemaphoreType.DMA,
        pltpu.SemaphoreType.DMA,
    ],
)


R = 1000


def _tc_b_body(x_ref, w_ref, deg_ref, g_ref):
    h = jnp.dot(x_ref[...], w_ref[...], preferred_element_type=jnp.float32)
    dis = lax.rsqrt(deg_ref[:, 0:1] + 1.0)
    g_ref[...] = dis * h


def _tc_d_body(pa_ref, pb_ref, g0_ref, deg0_ref, b0_ref, w1_ref, deg1_ref,
               g1_ref):
    ssum = (pa_ref[0] + pa_ref[1]) + (pb_ref[0] + pb_ref[1]) + g0_ref[...]
    dis0 = lax.rsqrt(deg0_ref[:, 0:1] + 1.0)
    a = jnp.maximum(dis0 * ssum + b0_ref[...], 0.0)
    h1 = jnp.dot(a, w1_ref[...], preferred_element_type=jnp.float32)
    dis1 = lax.rsqrt(deg1_ref[:, 0:1] + 1.0)
    g1_ref[...] = dis1 * h1


def _tc_f_body(pa_ref, pb_ref, g1_ref, deg1_ref, b1_ref, o_ref):
    dis1 = lax.rsqrt(deg1_ref[:, 0:1] + 1.0)
    v = dis1 * ((pa_ref[0] + pa_ref[1]) + (pb_ref[0] + pb_ref[1])
                + g1_ref[...]) + b1_ref[...]
    nrm = jnp.sqrt(jnp.sum(v * v, axis=1, keepdims=True))
    o_ref[...] = v / jnp.maximum(nrm, 1e-12)


def _row_spec(i):
    return (i, 0)


_tc_b = pl.pallas_call(
    _tc_b_body,
    grid=(N // R,),
    in_specs=[
        pl.BlockSpec((R, D), _row_spec),
        pl.BlockSpec((D, D), lambda i: (0, 0)),
        pl.BlockSpec((R, 16), _row_spec),
    ],
    out_specs=pl.BlockSpec((R, D), _row_spec),
    out_shape=jax.ShapeDtypeStruct((N, D), jnp.float32),
)

_part_spec = pl.BlockSpec((2, R, D), lambda i: (0, i, 0))

_tc_d = pl.pallas_call(
    _tc_d_body,
    grid=(N // R,),
    in_specs=[
        _part_spec,
        _part_spec,
        pl.BlockSpec((R, D), _row_spec),
        pl.BlockSpec((R, 16), _row_spec),
        pl.BlockSpec((1, D), lambda i: (0, 0)),
        pl.BlockSpec((D, D), lambda i: (0, 0)),
        pl.BlockSpec((R, 16), _row_spec),
    ],
    out_specs=pl.BlockSpec((R, D), _row_spec),
    out_shape=jax.ShapeDtypeStruct((N, D), jnp.float32),
)

_tc_f = pl.pallas_call(
    _tc_f_body,
    grid=(N // R,),
    in_specs=[
        _part_spec,
        _part_spec,
        pl.BlockSpec((R, D), _row_spec),
        pl.BlockSpec((R, 16), _row_spec),
        pl.BlockSpec((1, D), lambda i: (0, 0)),
    ],
    out_specs=pl.BlockSpec((R, D), _row_spec),
    out_shape=jax.ShapeDtypeStruct((N, D), jnp.float32),
)


@jax.jit
def kernel(x, edge_index_list, W0, b0, W1, b1):
    ei = edge_index_list.astype(jnp.int32)

    def pad_idx(a, fill):
        a2 = a.reshape(NW, EPW)
        pad = jnp.full((NW, APAD), fill, jnp.int32)
        return jnp.concatenate([a2, pad], axis=1).reshape(NW * AROWS, ACH)

    h = E // 2
    halves = [
        (pad_idx(ei[l, 0, i * h:(i + 1) * h], 0),
         pad_idx(ei[l, 1, i * h:(i + 1) * h], N))
        for l in range(2) for i in range(2)
    ]

    dsts_deg = jnp.stack([ei[0, 1], ei[1, 1]]).reshape(2, NS * DEG_TROWS, CH)
    ones_in = jnp.ones((CH, 16), jnp.float32)
    zeros16 = jnp.zeros((RPS, 16), jnp.float32)

    deg = _deg_kernel(dsts_deg, ones_in, zeros16)
    g0 = _tc_b(x, W0, deg[0])
    p0a = _agg_kernel(halves[0][0], halves[0][1], g0)
    p0b = _agg_kernel(halves[1][0], halves[1][1], g0)
    g1 = _tc_d(p0a, p0b, g0, deg[0], b0.reshape(1, D), W1, deg[1])
    p1a = _agg_kernel(halves[2][0], halves[2][1], g1)
    p1b = _agg_kernel(halves[3][0], halves[3][1], g1)
    return _tc_f(p1a, p1b, g1, deg[1], b1.reshape(1, D))
